# Initial kernel scaffold; baseline (speedup 1.0000x reference)
import jax
import jax.numpy as jnp
from jax.experimental import pallas as pl


def kernel(x, gate_w, q_w, k_w, v_w, b_w, a_w, g_w, o_w, A_log, dt_bias, o_norm_weight):
    raise NotImplementedError("write your pallas kernel here")


# speedup vs baseline: 1.6836x; 1.6836x over previous
import functools

import jax
import jax.numpy as jnp
from jax.experimental import pallas as pl
from jax.experimental.pallas import tpu as pltpu

B, T, HID = 2, 256, 1024
H, DK, M, TOPK = 4, 64, 8, 2
KD = H * DK
VD = 2 * KD
DV = VD // H
BT = B * T

_F32 = jnp.float32


def _silu(x):
    return x * jax.nn.sigmoid(x)


def _dense_stage(x_ref, gate_w_ref, q_w_ref, k_w_ref, v_w_ref, b_w_ref,
                 a_w_ref, g_w_ref, A_log_ref, dt_bias_ref,
                 qn_ref, kn_ref, vl_ref, gl_ref,
                 sel_ref, rw_ref, dec_ref, beta_ref):
    x2 = x_ref[...].reshape(BT, HID)

    logits = jnp.dot(x2, gate_w_ref[...], preferred_element_type=_F32)
    mx = jnp.max(logits, axis=1, keepdims=True)
    e = jnp.exp(logits - mx)
    s = e / jnp.sum(e, axis=1, keepdims=True)
    lane = jax.lax.broadcasted_iota(jnp.int32, (BT, M), 1)
    m1 = jnp.max(s, axis=1, keepdims=True)
    i1 = jnp.min(jnp.where(s == m1, lane, M), axis=1, keepdims=True)
    s2 = jnp.where(lane == i1, -1.0, s)
    m2 = jnp.max(s2, axis=1, keepdims=True)
    i2 = jnp.min(jnp.where(s2 == m2, lane, M), axis=1, keepdims=True)
    denom = m1 + m2
    sel_ref[...] = jnp.concatenate([i1, i2], axis=1).reshape(B, T, TOPK)
    rw_ref[...] = jnp.concatenate([m1 / denom, m2 / denom], axis=1).reshape(B, T, TOPK)

    beta = jax.nn.sigmoid(jnp.dot(x2, b_w_ref[...], preferred_element_type=_F32))
    beta_ref[...] = beta.reshape(B, T, H)
    a = jnp.dot(x2, a_w_ref[...], preferred_element_type=_F32) + dt_bias_ref[...]
    sp = jnp.maximum(a, 0.0) + jnp.log1p(jnp.exp(-jnp.abs(a)))
    dec_ref[...] = jnp.exp(-jnp.exp(A_log_ref[...]) * sp).reshape(B, T, H)

    qs = _silu(jnp.dot(x2, q_w_ref[...], preferred_element_type=_F32))
    ks = _silu(jnp.dot(x2, k_w_ref[...], preferred_element_type=_F32))
    vs = _silu(jnp.dot(x2, v_w_ref[...], preferred_element_type=_F32))
    gs = jnp.dot(x2, g_w_ref[...], preferred_element_type=_F32)
    scale = DK ** -0.5
    for hh in range(H):
        qh = qs[:, hh * DK:(hh + 1) * DK]
        nq = jnp.sqrt(jnp.sum(qh * qh, axis=1, keepdims=True))
        qn_ref[:, hh] = (qh / jnp.maximum(nq, 1e-12) * scale).reshape(B, T, DK)
        kh = ks[:, hh * DK:(hh + 1) * DK]
        nk = jnp.sqrt(jnp.sum(kh * kh, axis=1, keepdims=True))
        kn_ref[:, hh] = (kh / jnp.maximum(nk, 1e-12)).reshape(B, T, DK)
        vl_ref[:, hh] = vs[:, hh * DV:(hh + 1) * DV].reshape(B, T, DV)
        gl_ref[:, hh] = gs[:, hh * DV:(hh + 1) * DV].reshape(B, T, DV)


def _scan_stage(qn_ref, kn_ref, vl_ref, sel_ref, rw_ref, dec_ref, beta_ref,
                oc_ref, h_ref):
    h_ref[...] = jnp.zeros((M * B, H, DK, DV), dtype=_F32)

    def step(t, carry):
        for b in range(B):
            dec_b = [dec_ref[b, pl.ds(t, 1), hh:hh + 1] for hh in range(H)]
            beta_b = [beta_ref[b, pl.ds(t, 1), hh:hh + 1] for hh in range(H)]
            k_rows = [kn_ref[b, hh, pl.ds(t, 1), :] for hh in range(H)]
            q_rows = [qn_ref[b, hh, pl.ds(t, 1), :] for hh in range(H)]
            v_rows = [vl_ref[b, hh, pl.ds(t, 1), :] for hh in range(H)]
            acc = [jnp.zeros((1, DV), dtype=_F32) for _ in range(H)]
            for slot in range(TOPK):
                mid = sel_ref[b, t, slot]
                w = rw_ref[b, pl.ds(t, 1), slot:slot + 1]
                idx = mid * B + b
                for hh in range(H):
                    hm = h_ref[idx, hh] * dec_b[hh]
                    pred = jax.lax.dot_general(
                        k_rows[hh], hm, (((1,), (0,)), ((), ())),
                        preferred_element_type=_F32)
                    vnew = beta_b[hh] * (v_rows[hh] - pred)
                    hm = hm + jax.lax.dot_general(
                        k_rows[hh], vnew, (((0,), (0,)), ((), ())),
                        preferred_element_type=_F32)
                    h_ref[idx, hh] = hm
                    o = jax.lax.dot_general(
                        q_rows[hh], hm, (((1,), (0,)), ((), ())),
                        preferred_element_type=_F32)
                    acc[hh] = acc[hh] + w * o
            for hh in range(H):
                oc_ref[b, hh, pl.ds(t, 1), :] = acc[hh]
        return carry

    jax.lax.fori_loop(0, T, step, 0, unroll=False)


def _out_stage(oc_ref, gl_ref, o_w_ref, onw_ref, out_ref):
    for b in range(B):
        acc = jnp.zeros((T, HID), dtype=_F32)
        for hh in range(H):
            y = oc_ref[b, hh]
            rms = jnp.sqrt(jnp.mean(y * y, axis=1, keepdims=True) + 1e-6)
            srow = (y / rms) * onw_ref[...] * jax.nn.sigmoid(gl_ref[b, hh])
            acc = acc + jnp.dot(srow, o_w_ref[hh * DV:(hh + 1) * DV, :],
                                preferred_element_type=_F32)
        out_ref[b] = acc


def _vmem():
    return pl.BlockSpec(memory_space=pltpu.VMEM)


def _smem():
    return pl.BlockSpec(memory_space=pltpu.SMEM)


@jax.jit
def kernel(x, gate_w, q_w, k_w, v_w, b_w, a_w, g_w, o_w, A_log, dt_bias,
           o_norm_weight):
    A_log2 = A_log.reshape(1, H)
    dt2 = dt_bias.reshape(1, H)
    onw2 = o_norm_weight.reshape(1, DV)

    qn, kn, vl, gl, sel, rw, dec, beta = pl.pallas_call(
        _dense_stage,
        in_specs=[_vmem()] * 10,
        out_specs=(_vmem(), _vmem(), _vmem(), _vmem(),
                   _vmem(), _vmem(), _vmem(), _vmem()),
        out_shape=(
            jax.ShapeDtypeStruct((B, H, T, DK), _F32),
            jax.ShapeDtypeStruct((B, H, T, DK), _F32),
            jax.ShapeDtypeStruct((B, H, T, DV), _F32),
            jax.ShapeDtypeStruct((B, H, T, DV), _F32),
            jax.ShapeDtypeStruct((B, T, TOPK), jnp.int32),
            jax.ShapeDtypeStruct((B, T, TOPK), _F32),
            jax.ShapeDtypeStruct((B, T, H), _F32),
            jax.ShapeDtypeStruct((B, T, H), _F32),
        ),
    )(x, gate_w, q_w, k_w, v_w, b_w, a_w, g_w, A_log2, dt2)

    oc = pl.pallas_call(
        _scan_stage,
        in_specs=[_vmem(), _vmem(), _vmem(), _smem(), _vmem(), _vmem(), _vmem()],
        out_specs=_vmem(),
        out_shape=jax.ShapeDtypeStruct((B, H, T, DV), _F32),
        scratch_shapes=[pltpu.VMEM((M * B, H, DK, DV), _F32)],
    )(qn, kn, vl, sel, rw, dec, beta)

    out = pl.pallas_call(
        _out_stage,
        in_specs=[_vmem(), _vmem(), _vmem(), _vmem()],
        out_specs=_vmem(),
        out_shape=jax.ShapeDtypeStruct((B, T, HID), _F32),
    )(oc, gl, o_w, onw2)
    return out


# speedup vs baseline: 3.4780x; 2.0658x over previous
import functools

import jax
import jax.numpy as jnp
from jax.experimental import pallas as pl
from jax.experimental.pallas import tpu as pltpu

B, T, HID = 2, 256, 1024
H, DK, M, TOPK = 4, 64, 8, 2
KD = H * DK
VD = 2 * KD
DV = VD // H
BT = B * T

_F32 = jnp.float32


def _silu(x):
    return x * jax.nn.sigmoid(x)


def _dense_stage(x_ref, gate_w_ref, q_w_ref, k_w_ref, v_w_ref, b_w_ref,
                 a_w_ref, g_w_ref, A_log_ref, dt_bias_ref,
                 qn_ref, kn_ref, vl_ref, gl_ref,
                 sel_ref, rw_ref, dec_ref, beta_ref):
    x2 = x_ref[...].reshape(BT, HID)

    logits = jnp.dot(x2, gate_w_ref[...], preferred_element_type=_F32)
    mx = jnp.max(logits, axis=1, keepdims=True)
    e = jnp.exp(logits - mx)
    s = e / jnp.sum(e, axis=1, keepdims=True)
    lane = jax.lax.broadcasted_iota(jnp.int32, (BT, M), 1)
    m1 = jnp.max(s, axis=1, keepdims=True)
    i1 = jnp.min(jnp.where(s == m1, lane, M), axis=1, keepdims=True)
    s2 = jnp.where(lane == i1, -1.0, s)
    m2 = jnp.max(s2, axis=1, keepdims=True)
    i2 = jnp.min(jnp.where(s2 == m2, lane, M), axis=1, keepdims=True)
    denom = m1 + m2
    sel_ref[...] = jnp.concatenate([i1, i2], axis=1).reshape(B, T, TOPK)
    rw_ref[...] = jnp.concatenate([m1 / denom, m2 / denom], axis=1).reshape(B, T, TOPK)

    beta = jax.nn.sigmoid(jnp.dot(x2, b_w_ref[...], preferred_element_type=_F32))
    beta_ref[...] = beta.reshape(B, T, H)
    a = jnp.dot(x2, a_w_ref[...], preferred_element_type=_F32) + dt_bias_ref[...]
    sp = jnp.maximum(a, 0.0) + jnp.log1p(jnp.exp(-jnp.abs(a)))
    dec_ref[...] = jnp.exp(-jnp.exp(A_log_ref[...]) * sp).reshape(B, T, H)

    qs = _silu(jnp.dot(x2, q_w_ref[...], preferred_element_type=_F32))
    ks = _silu(jnp.dot(x2, k_w_ref[...], preferred_element_type=_F32))
    vs = _silu(jnp.dot(x2, v_w_ref[...], preferred_element_type=_F32))
    gs = jnp.dot(x2, g_w_ref[...], preferred_element_type=_F32)
    scale = DK ** -0.5
    for hh in range(H):
        qh = qs[:, hh * DK:(hh + 1) * DK]
        nq = jnp.sqrt(jnp.sum(qh * qh, axis=1, keepdims=True))
        qn_ref[:, hh] = (qh / jnp.maximum(nq, 1e-12) * scale).reshape(B, T, DK)
        kh = ks[:, hh * DK:(hh + 1) * DK]
        nk = jnp.sqrt(jnp.sum(kh * kh, axis=1, keepdims=True))
        kn_ref[:, hh] = (kh / jnp.maximum(nk, 1e-12)).reshape(B, T, DK)
        vl_ref[:, hh] = vs[:, hh * DV:(hh + 1) * DV].reshape(B, T, DV)
        gl_ref[:, hh] = gs[:, hh * DV:(hh + 1) * DV].reshape(B, T, DV)


def _scan_stage(qn_ref, kn_ref, vl_ref, sel_ref, rw_ref, dec_ref, beta_ref,
                oc_ref, h_ref):
    h_ref[...] = jnp.zeros((M * B, H, DK, DV), dtype=_F32)

    def _mv(row, mat):
        return jax.lax.dot_general(row, mat, (((1,), (0,)), ((), ())),
                                   preferred_element_type=_F32)

    def _outer(row, col):
        return jax.lax.dot_general(row, col, (((0,), (0,)), ((), ())),
                                   preferred_element_type=_F32)

    def step(t, carry):
        dec_b, beta_b, k_rows, q_rows, v_rows, qk_b = [], [], [], [], [], []
        idxs, ws, hms = [], [], []
        for b in range(B):
            dec_b.append([dec_ref[b, pl.ds(t, 1), hh:hh + 1] for hh in range(H)])
            beta_b.append([beta_ref[b, pl.ds(t, 1), hh:hh + 1] for hh in range(H)])
            k_rows.append([kn_ref[b, hh, pl.ds(t, 1), :] for hh in range(H)])
            q_rows.append([qn_ref[b, hh, pl.ds(t, 1), :] for hh in range(H)])
            v_rows.append([vl_ref[b, hh, pl.ds(t, 1), :] for hh in range(H)])
            qk_b.append([jnp.sum(q_rows[b][hh] * k_rows[b][hh], axis=1,
                                 keepdims=True) for hh in range(H)])
            idx_b, w_b, hm_b = [], [], []
            for slot in range(TOPK):
                idx = sel_ref[b, t, slot] * B + b
                idx_b.append(idx)
                w_b.append(rw_ref[b, pl.ds(t, 1), slot:slot + 1])
                hm_b.append([h_ref[idx, hh] for hh in range(H)])
            idxs.append(idx_b)
            ws.append(w_b)
            hms.append(hm_b)

        new_h = [[[None] * H for _ in range(TOPK)] for _ in range(B)]
        for b in range(B):
            acc = [jnp.zeros((1, DV), dtype=_F32) for _ in range(H)]
            for slot in range(TOPK):
                w = ws[b][slot]
                for hh in range(H):
                    hm = hms[b][slot][hh]
                    dec = dec_b[b][hh]
                    kh = _mv(k_rows[b][hh], hm)
                    qh = _mv(q_rows[b][hh], hm)
                    vnew = beta_b[b][hh] * (v_rows[b][hh] - dec * kh)
                    new_h[b][slot][hh] = dec * hm + _outer(k_rows[b][hh], vnew)
                    o = dec * qh + qk_b[b][hh] * vnew
                    acc[hh] = acc[hh] + w * o
            for hh in range(H):
                oc_ref[b, hh, pl.ds(t, 1), :] = acc[hh]
        for b in range(B):
            for slot in range(TOPK):
                for hh in range(H):
                    h_ref[idxs[b][slot], hh] = new_h[b][slot][hh]
        return carry

    jax.lax.fori_loop(0, T, step, 0, unroll=False)


def _out_stage(oc_ref, gl_ref, o_w_ref, onw_ref, out_ref):
    for b in range(B):
        acc = jnp.zeros((T, HID), dtype=_F32)
        for hh in range(H):
            y = oc_ref[b, hh]
            rms = jnp.sqrt(jnp.mean(y * y, axis=1, keepdims=True) + 1e-6)
            srow = (y / rms) * onw_ref[...] * jax.nn.sigmoid(gl_ref[b, hh])
            acc = acc + jnp.dot(srow, o_w_ref[hh * DV:(hh + 1) * DV, :],
                                preferred_element_type=_F32)
        out_ref[b] = acc


def _vmem():
    return pl.BlockSpec(memory_space=pltpu.VMEM)


def _smem():
    return pl.BlockSpec(memory_space=pltpu.SMEM)


@jax.jit
def kernel(x, gate_w, q_w, k_w, v_w, b_w, a_w, g_w, o_w, A_log, dt_bias,
           o_norm_weight):
    A_log2 = A_log.reshape(1, H)
    dt2 = dt_bias.reshape(1, H)
    onw2 = o_norm_weight.reshape(1, DV)

    qn, kn, vl, gl, sel, rw, dec, beta = pl.pallas_call(
        _dense_stage,
        in_specs=[_vmem()] * 10,
        out_specs=(_vmem(), _vmem(), _vmem(), _vmem(),
                   _vmem(), _vmem(), _vmem(), _vmem()),
        out_shape=(
            jax.ShapeDtypeStruct((B, H, T, DK), _F32),
            jax.ShapeDtypeStruct((B, H, T, DK), _F32),
            jax.ShapeDtypeStruct((B, H, T, DV), _F32),
            jax.ShapeDtypeStruct((B, H, T, DV), _F32),
            jax.ShapeDtypeStruct((B, T, TOPK), jnp.int32),
            jax.ShapeDtypeStruct((B, T, TOPK), _F32),
            jax.ShapeDtypeStruct((B, T, H), _F32),
            jax.ShapeDtypeStruct((B, T, H), _F32),
        ),
    )(x, gate_w, q_w, k_w, v_w, b_w, a_w, g_w, A_log2, dt2)

    oc = pl.pallas_call(
        _scan_stage,
        in_specs=[_vmem(), _vmem(), _vmem(), _smem(), _vmem(), _vmem(), _vmem()],
        out_specs=_vmem(),
        out_shape=jax.ShapeDtypeStruct((B, H, T, DV), _F32),
        scratch_shapes=[pltpu.VMEM((M * B, H, DK, DV), _F32)],
    )(qn, kn, vl, sel, rw, dec, beta)

    out = pl.pallas_call(
        _out_stage,
        in_specs=[_vmem(), _vmem(), _vmem(), _vmem()],
        out_specs=_vmem(),
        out_shape=jax.ShapeDtypeStruct((B, T, HID), _F32),
    )(oc, gl, o_w, onw2)
    return out


# speedup vs baseline: 5.4912x; 1.5788x over previous
import jax
import jax.numpy as jnp
from jax.experimental import pallas as pl
from jax.experimental.pallas import tpu as pltpu

B, T, HID = 2, 256, 1024
H, DK, M, TOPK = 4, 64, 8, 2
KD = H * DK
VD = 2 * KD
DV = VD // H
BT = B * T
SH = TOPK * H
SKD = TOPK * H * DK

_F32 = jnp.float32


def _silu(x):
    return x * jax.nn.sigmoid(x)


def _dense_stage(x_ref, gate_w_ref, q_w_ref, k_w_ref, v_w_ref, b_w_ref,
                 a_w_ref, g_w_ref, A_log_ref, dt_bias_ref,
                 kbd_ref, qbd_ref, vb_ref, gl_ref,
                 sel_ref, wdec_ref, wqk_ref, bd_ref, dec_ref):
    x2 = x_ref[...].reshape(BT, HID)

    logits = jnp.dot(x2, gate_w_ref[...], preferred_element_type=_F32)
    mx = jnp.max(logits, axis=1, keepdims=True)
    e = jnp.exp(logits - mx)
    s = e / jnp.sum(e, axis=1, keepdims=True)
    lane = jax.lax.broadcasted_iota(jnp.int32, (BT, M), 1)
    m1 = jnp.max(s, axis=1, keepdims=True)
    i1 = jnp.min(jnp.where(s == m1, lane, M), axis=1, keepdims=True)
    s2 = jnp.where(lane == i1, -1.0, s)
    m2 = jnp.max(s2, axis=1, keepdims=True)
    i2 = jnp.min(jnp.where(s2 == m2, lane, M), axis=1, keepdims=True)
    denom = m1 + m2
    sel_ref[...] = jnp.concatenate([i1, i2], axis=1).reshape(B, T, TOPK)
    rw = [m1 / denom, m2 / denom]

    beta = jax.nn.sigmoid(jnp.dot(x2, b_w_ref[...], preferred_element_type=_F32))
    a = jnp.dot(x2, a_w_ref[...], preferred_element_type=_F32) + dt_bias_ref[...]
    sp = jnp.maximum(a, 0.0) + jnp.log1p(jnp.exp(-jnp.abs(a)))
    dec = jnp.exp(-jnp.exp(A_log_ref[...]) * sp)
    dec_ref[...] = dec.reshape(B, T, H)
    bd_ref[...] = (beta * dec).reshape(B, T, H)

    qs = _silu(jnp.dot(x2, q_w_ref[...], preferred_element_type=_F32))
    ks = _silu(jnp.dot(x2, k_w_ref[...], preferred_element_type=_F32))
    vs = _silu(jnp.dot(x2, v_w_ref[...], preferred_element_type=_F32))
    gs = jnp.dot(x2, g_w_ref[...], preferred_element_type=_F32)
    scale = DK ** -0.5

    kbd_ref[...] = jnp.zeros((B, T, SH, SKD), dtype=_F32)
    qbd_ref[...] = jnp.zeros((B, T, SH, SKD), dtype=_F32)
    qk_cols = []
    for hh in range(H):
        qh = qs[:, hh * DK:(hh + 1) * DK]
        nq = jnp.sqrt(jnp.sum(qh * qh, axis=1, keepdims=True))
        qn = qh / jnp.maximum(nq, 1e-12) * scale
        kh = ks[:, hh * DK:(hh + 1) * DK]
        nk = jnp.sqrt(jnp.sum(kh * kh, axis=1, keepdims=True))
        kn = kh / jnp.maximum(nk, 1e-12)
        qk_cols.append(jnp.sum(qn * kn, axis=1, keepdims=True))
        vb_ref[:, hh] = (beta[:, hh:hh + 1] * vs[:, hh * DV:(hh + 1) * DV]
                         ).reshape(B, T, DV)
        gl_ref[:, hh] = gs[:, hh * DV:(hh + 1) * DV].reshape(B, T, DV)
        for slot in range(TOPK):
            r = slot * H + hh
            c = slot * KD + hh * DK
            kbd_ref[:, :, r, c:c + DK] = kn.reshape(B, T, DK)
            qbd_ref[:, :, r, c:c + DK] = qn.reshape(B, T, DK)

    qk = jnp.concatenate(qk_cols, axis=1)
    wdec = jnp.concatenate([rw[s_] * dec for s_ in range(TOPK)], axis=1)
    wqk = jnp.concatenate([rw[s_] * qk for s_ in range(TOPK)], axis=1)
    wdec_ref[...] = wdec.reshape(B, T, SH)
    wqk_ref[...] = wqk.reshape(B, T, SH)


def _scan_stage(kbd_ref, qbd_ref, vb_ref, sel_ref, wdec_ref, wqk_ref,
                bd_ref, dec_ref, oc_ref, h_ref):
    h_ref[...] = jnp.zeros((M * B, KD, DV), dtype=_F32)

    def step(t, carry):
        for b in range(B):
            i0 = sel_ref[b, t, 0] * B + b
            i1 = sel_ref[b, t, 1] * B + b
            hp = jnp.concatenate([h_ref[i0], h_ref[i1]], axis=0)
            kb = kbd_ref[b, pl.ds(t, 1)].reshape(SH, SKD)
            qb = qbd_ref[b, pl.ds(t, 1)].reshape(SH, SKD)
            pred = jax.lax.dot_general(kb, hp, (((1,), (0,)), ((), ())),
                                       preferred_element_type=_F32)
            qh = jax.lax.dot_general(qb, hp, (((1,), (0,)), ((), ())),
                                     preferred_element_type=_F32)
            vnew_rows = []
            o_rows = []
            for slot in range(TOPK):
                for hh in range(H):
                    r = slot * H + hh
                    bd = bd_ref[b, pl.ds(t, 1), hh:hh + 1]
                    wd = wdec_ref[b, pl.ds(t, 1), r:r + 1]
                    wq = wqk_ref[b, pl.ds(t, 1), r:r + 1]
                    vrow = vb_ref[b, hh, pl.ds(t, 1), :]
                    vnew = vrow - bd * pred[r:r + 1]
                    vnew_rows.append(vnew)
                    o_rows.append(wd * qh[r:r + 1] + wq * vnew)
            vnew8 = jnp.concatenate(vnew_rows, axis=0)
            outer = jax.lax.dot_general(kb, vnew8, (((0,), (0,)), ((), ())),
                                        preferred_element_type=_F32)
            acc = jnp.concatenate(
                [o_rows[hh] + o_rows[H + hh] for hh in range(H)], axis=0)
            oc_ref[b, pl.ds(t, 1)] = acc.reshape(1, H, DV)
            for slot in range(TOPK):
                idx = i0 if slot == 0 else i1
                base = slot * KD
                for hh in range(H):
                    dec = dec_ref[b, pl.ds(t, 1), hh:hh + 1]
                    blk = hp[base + hh * DK:base + (hh + 1) * DK] * dec
                    h_ref[idx, hh * DK:(hh + 1) * DK] = (
                        blk + outer[base + hh * DK:base + (hh + 1) * DK])
        return carry

    jax.lax.fori_loop(0, T, step, 0, unroll=False)


def _out_stage(oc_ref, gl_ref, o_w_ref, onw_ref, out_ref):
    for b in range(B):
        acc = jnp.zeros((T, HID), dtype=_F32)
        for hh in range(H):
            y = oc_ref[b, :, hh, :]
            rms = jnp.sqrt(jnp.mean(y * y, axis=1, keepdims=True) + 1e-6)
            srow = (y / rms) * onw_ref[...] * jax.nn.sigmoid(gl_ref[b, hh])
            acc = acc + jnp.dot(srow, o_w_ref[hh * DV:(hh + 1) * DV, :],
                                preferred_element_type=_F32)
        out_ref[b] = acc


def _vmem():
    return pl.BlockSpec(memory_space=pltpu.VMEM)


def _smem():
    return pl.BlockSpec(memory_space=pltpu.SMEM)


@jax.jit
def kernel(x, gate_w, q_w, k_w, v_w, b_w, a_w, g_w, o_w, A_log, dt_bias,
           o_norm_weight):
    A_log2 = A_log.reshape(1, H)
    dt2 = dt_bias.reshape(1, H)
    onw2 = o_norm_weight.reshape(1, DV)

    kbd, qbd, vb, gl, sel, wdec, wqk, bd, dec = pl.pallas_call(
        _dense_stage,
        in_specs=[_vmem()] * 10,
        out_specs=(_vmem(),) * 9,
        out_shape=(
            jax.ShapeDtypeStruct((B, T, SH, SKD), _F32),
            jax.ShapeDtypeStruct((B, T, SH, SKD), _F32),
            jax.ShapeDtypeStruct((B, H, T, DV), _F32),
            jax.ShapeDtypeStruct((B, H, T, DV), _F32),
            jax.ShapeDtypeStruct((B, T, TOPK), jnp.int32),
            jax.ShapeDtypeStruct((B, T, SH), _F32),
            jax.ShapeDtypeStruct((B, T, SH), _F32),
            jax.ShapeDtypeStruct((B, T, H), _F32),
            jax.ShapeDtypeStruct((B, T, H), _F32),
        ),
    )(x, gate_w, q_w, k_w, v_w, b_w, a_w, g_w, A_log2, dt2)

    oc = pl.pallas_call(
        _scan_stage,
        in_specs=[_vmem(), _vmem(), _vmem(), _smem(),
                  _vmem(), _vmem(), _vmem(), _vmem()],
        out_specs=_vmem(),
        out_shape=jax.ShapeDtypeStruct((B, T, H, DV), _F32),
        scratch_shapes=[pltpu.VMEM((M * B, KD, DV), _F32)],
    )(kbd, qbd, vb, sel, wdec, wqk, bd, dec)

    out = pl.pallas_call(
        _out_stage,
        in_specs=[_vmem(), _vmem(), _vmem(), _vmem()],
        out_specs=_vmem(),
        out_shape=jax.ShapeDtypeStruct((B, T, HID), _F32),
    )(oc, gl, o_w, onw2)
    return out
